# baseline (speedup 1.0000x reference)
import functools

import jax
import jax.numpy as jnp
from jax.experimental import pallas as pl
from jax.experimental.pallas import tpu as pltpu
from jax.experimental.pallas import tpu_sc as plsc

N_NODES = 10000
N_EDGES = 320000
D = 128

NC = 2
NS = 16
NW = NC * NS
NP = 10240
CH = 80
EPW = CH * 128
RPT = NP // NS
SB = 40
CW = NW * CH // NS

_mesh = plsc.VectorSubcoreMesh(
    core_axis_name="c", subcore_axis_name="s", num_cores=NC, num_subcores=NS)


def _deg_body(srcr, dstr, zeros_np, ones_hbm, dsrc_out, ddst_out,
              sidx, didx, ones_v, hs, hd):
    c = jax.lax.axis_index("c")
    s = jax.lax.axis_index("s")
    w = s * NC + c
    pltpu.sync_copy(srcr.at[w], sidx)
    pltpu.sync_copy(dstr.at[w], didx)
    pltpu.sync_copy(ones_hbm, ones_v)

    @pl.when(s == 0)
    def _():
        pltpu.sync_copy(zeros_np, hs)
        pltpu.sync_copy(zeros_np, hd)

    plsc.subcore_barrier()

    def body(j, carry):
        pltpu.sync_copy(ones_v, hs.at[sidx.at[j]], add=True)
        pltpu.sync_copy(ones_v, hd.at[didx.at[j]], add=True)
        return carry

    jax.lax.fori_loop(0, CH, body, 0)
    plsc.subcore_barrier()

    @pl.when(s == 0)
    def _():
        pltpu.sync_copy(hs, dsrc_out.at[c])
        pltpu.sync_copy(hd, ddst_out.at[c])


_deg_call = functools.partial(
    pl.kernel,
    out_type=[
        jax.ShapeDtypeStruct((NC, NP), jnp.float32),
        jax.ShapeDtypeStruct((NC, NP), jnp.float32),
    ],
    mesh=_mesh,
    scratch_types=[
        pltpu.VMEM((CH, 128), jnp.int32),
        pltpu.VMEM((CH, 128), jnp.int32),
        pltpu.VMEM((128,), jnp.float32),
        pltpu.VMEM_SHARED((NP,), jnp.float32),
        pltpu.VMEM_SHARED((NP,), jnp.float32),
    ],
)(_deg_body)


def _agg_body(h, srcr, dstr, zrows, out, sblk, dblk, rows0, rows1, aggsh,
              sem0, sem1):
    c = jax.lax.axis_index("c")
    s = jax.lax.axis_index("s")

    @pl.when(c == 0)
    def _():
        pltpu.sync_copy(zrows, aggsh.at[pl.ds(s * RPT, RPT)])
        plsc.subcore_barrier()

        def wait_gather(buf, sem):
            pltpu.make_async_copy(h.at[pl.ds(0, 128)], buf, sem).wait()

        def do_superchunk(row0):
            pltpu.sync_copy(srcr.at[pl.ds(row0, SB)], sblk)
            pltpu.sync_copy(dstr.at[pl.ds(row0, SB)], dblk)
            pltpu.async_copy(h.at[sblk.at[0]], rows0, sem0)

            def inner(i, carry2):
                j0 = 2 * i
                j1 = j0 + 1
                pltpu.async_copy(h.at[sblk.at[j1]], rows1, sem1)
                wait_gather(rows0, sem0)
                pltpu.sync_copy(rows0, aggsh.at[dblk.at[j0]], add=True)

                @pl.when(i < SB // 2 - 1)
                def _():
                    pltpu.async_copy(h.at[sblk.at[j0 + 2]], rows0, sem0)

                wait_gather(rows1, sem1)
                pltpu.sync_copy(rows1, aggsh.at[dblk.at[j1]], add=True)
                return carry2

            jax.lax.fori_loop(0, SB // 2, inner, 0)

        def outer(b, carry):
            do_superchunk(s * CW + b * SB)
            return carry

        jax.lax.fori_loop(0, CW // SB, outer, 0)
        plsc.subcore_barrier()
        pltpu.sync_copy(aggsh.at[pl.ds(s * RPT, RPT)],
                        out.at[pl.ds(s * RPT, RPT)])


_agg_call = functools.partial(
    pl.kernel,
    out_type=jax.ShapeDtypeStruct((NP, D), jnp.float32),
    mesh=_mesh,
    scratch_types=[
        pltpu.VMEM((SB, 128), jnp.int32),
        pltpu.VMEM((SB, 128), jnp.int32),
        pltpu.VMEM((128, D), jnp.float32),
        pltpu.VMEM((128, D), jnp.float32),
        pltpu.VMEM_SHARED((NP, D), jnp.float32),
        pltpu.SemaphoreType.DMA,
        pltpu.SemaphoreType.DMA,
    ],
)(_agg_body)


def _prep_body(x_ref, ds_ref, dd_ref, h_ref, nd_ref):
    ds = jnp.sum(ds_ref[...], axis=0)
    dd = jnp.sum(dd_ref[...], axis=0)
    ns = jax.lax.rsqrt(jnp.maximum(ds, 1.0))
    h_ref[...] = x_ref[...] * ns[:, None]
    nd_ref[...] = jax.lax.rsqrt(jnp.maximum(dd, 1.0)).reshape(NP, 1)


def _head_body(p_ref, nd_ref, wg_ref, bg_ref, wc_ref, bc_ref, ws_ref, bs_ref,
               z_ref, y_ref, s_ref):
    agg = p_ref[...] * nd_ref[...]
    z = jnp.dot(agg, wg_ref[...], preferred_element_type=jnp.float32)
    z = jnp.maximum(z + bg_ref[...], 0.0)
    z_ref[...] = z
    y_ref[...] = jnp.dot(z, wc_ref[...],
                         preferred_element_type=jnp.float32) + bc_ref[...]
    s_ref[...] = jnp.dot(z, ws_ref[...],
                         preferred_element_type=jnp.float32) + bs_ref[...]


def kernel(x, edge_index, W_gcn, b_gcn, W_cls, b_cls, W_sen, b_sen):
    src = edge_index[0]
    dst = edge_index[1]
    pad = NW * EPW - N_EDGES
    padv = jnp.full((pad,), N_NODES, dtype=jnp.int32)
    srcr = jnp.concatenate([src, padv]).reshape(NW, CH, 128)
    dstr = jnp.concatenate([dst, padv]).reshape(NW, CH, 128)
    x_pad = jnp.pad(x, ((0, NP - N_NODES), (0, 0)))
    zeros_np = jnp.zeros((NP,), jnp.float32)
    zeros_rows = jnp.zeros((RPT, D), jnp.float32)
    ones_128 = jnp.ones((128,), jnp.float32)

    dsrc_p, ddst_p = _deg_call(srcr, dstr, zeros_np, ones_128)

    h, nd = pl.pallas_call(
        _prep_body,
        out_shape=[
            jax.ShapeDtypeStruct((NP, D), jnp.float32),
            jax.ShapeDtypeStruct((NP, 1), jnp.float32),
        ],
    )(x_pad, dsrc_p, ddst_p)

    parts = _agg_call(h, srcr.reshape(NW * CH, 128),
                      dstr.reshape(NW * CH, 128), zeros_rows)

    z, y, s = pl.pallas_call(
        _head_body,
        out_shape=[
            jax.ShapeDtypeStruct((NP, D), jnp.float32),
            jax.ShapeDtypeStruct((NP, 1), jnp.float32),
            jax.ShapeDtypeStruct((NP, 1), jnp.float32),
        ],
    )(parts, nd, W_gcn, b_gcn.reshape(1, D), W_cls, b_cls.reshape(1, 1),
      W_sen, b_sen.reshape(1, 1))

    return (z[:N_NODES], y[:N_NODES], s[:N_NODES])


# speedup vs baseline: 2.9012x; 2.9012x over previous
import functools

import jax
import jax.numpy as jnp
from jax.experimental import pallas as pl
from jax.experimental.pallas import tpu as pltpu
from jax.experimental.pallas import tpu_sc as plsc

N_NODES = 10000
N_EDGES = 320000
D = 128

NC = 2
NS = 16
NW = NC * NS
NP = 10240
CH = 80
EPW = CH * 128
RPT = NP // NS
SB = 40

_mesh = plsc.VectorSubcoreMesh(
    core_axis_name="c", subcore_axis_name="s", num_cores=NC, num_subcores=NS)


def _deg_body(srcr, dstr, zeros_np, ones_hbm, dsrc_out, ddst_out,
              sidx, didx, ones_v, hs, hd):
    c = jax.lax.axis_index("c")
    s = jax.lax.axis_index("s")
    w = s * NC + c
    pltpu.sync_copy(srcr.at[w], sidx)
    pltpu.sync_copy(dstr.at[w], didx)
    pltpu.sync_copy(ones_hbm, ones_v)

    @pl.when(s == 0)
    def _():
        pltpu.sync_copy(zeros_np, hs)
        pltpu.sync_copy(zeros_np, hd)

    plsc.subcore_barrier()

    def body(j, carry):
        pltpu.sync_copy(ones_v, hs.at[sidx.at[j]], add=True)
        pltpu.sync_copy(ones_v, hd.at[didx.at[j]], add=True)
        return carry

    jax.lax.fori_loop(0, CH, body, 0)
    plsc.subcore_barrier()

    @pl.when(s == 0)
    def _():
        pltpu.sync_copy(hs, dsrc_out.at[c])
        pltpu.sync_copy(hd, ddst_out.at[c])


_deg_call = functools.partial(
    pl.kernel,
    out_type=[
        jax.ShapeDtypeStruct((NC, NP), jnp.float32),
        jax.ShapeDtypeStruct((NC, NP), jnp.float32),
    ],
    mesh=_mesh,
    scratch_types=[
        pltpu.VMEM((CH, 128), jnp.int32),
        pltpu.VMEM((CH, 128), jnp.int32),
        pltpu.VMEM((128,), jnp.float32),
        pltpu.VMEM_SHARED((NP,), jnp.float32),
        pltpu.VMEM_SHARED((NP,), jnp.float32),
    ],
)(_deg_body)


def _agg_body(h, srcr, dstr, zrows, out, sblk, dblk, rows0, rows1, aggsh,
              sem0, sem1):
    c = jax.lax.axis_index("c")
    s = jax.lax.axis_index("s")
    pltpu.sync_copy(zrows, aggsh.at[pl.ds(s * RPT, RPT)])
    plsc.subcore_barrier()

    base = (c * NS + s) * CH

    def wait_gather(buf, sem):
        pltpu.make_async_copy(h.at[pl.ds(0, 128)], buf, sem).wait()

    def do_superchunk(row0):
        pltpu.sync_copy(srcr.at[pl.ds(row0, SB)], sblk)
        pltpu.sync_copy(dstr.at[pl.ds(row0, SB)], dblk)
        pltpu.async_copy(h.at[sblk.at[0]], rows0, sem0)

        def inner(i, carry2):
            j0 = 2 * i
            j1 = j0 + 1
            pltpu.async_copy(h.at[sblk.at[j1]], rows1, sem1)
            wait_gather(rows0, sem0)
            pltpu.sync_copy(rows0, aggsh.at[dblk.at[j0]], add=True)

            @pl.when(i < SB // 2 - 1)
            def _():
                pltpu.async_copy(h.at[sblk.at[j0 + 2]], rows0, sem0)

            wait_gather(rows1, sem1)
            pltpu.sync_copy(rows1, aggsh.at[dblk.at[j1]], add=True)
            return carry2

        jax.lax.fori_loop(0, SB // 2, inner, 0)

    def outer(b, carry):
        do_superchunk(base + b * SB)
        return carry

    jax.lax.fori_loop(0, CH // SB, outer, 0)
    plsc.subcore_barrier()
    pltpu.sync_copy(aggsh.at[pl.ds(s * RPT, RPT)],
                    out.at[c, pl.ds(s * RPT, RPT)])


_agg_call = functools.partial(
    pl.kernel,
    out_type=jax.ShapeDtypeStruct((NC, NP, D), jnp.float32),
    mesh=_mesh,
    scratch_types=[
        pltpu.VMEM((SB, 128), jnp.int32),
        pltpu.VMEM((SB, 128), jnp.int32),
        pltpu.VMEM((128, D), jnp.float32),
        pltpu.VMEM((128, D), jnp.float32),
        pltpu.VMEM_SHARED((NP, D), jnp.float32),
        pltpu.SemaphoreType.DMA,
        pltpu.SemaphoreType.DMA,
    ],
)(_agg_body)


def _prep_body(x_ref, ds_ref, dd_ref, h_ref, nd_ref):
    ds = jnp.sum(ds_ref[...], axis=0)
    dd = jnp.sum(dd_ref[...], axis=0)
    ns = jax.lax.rsqrt(jnp.maximum(ds, 1.0))
    h_ref[...] = x_ref[...] * ns[:, None]
    nd_ref[...] = jax.lax.rsqrt(jnp.maximum(dd, 1.0)).reshape(NP, 1)


def _head_body(p_ref, nd_ref, wg_ref, bg_ref, wc_ref, bc_ref, ws_ref, bs_ref,
               z_ref, y_ref, s_ref):
    agg = (p_ref[0] + p_ref[1]) * nd_ref[...]
    z = jnp.dot(agg, wg_ref[...], preferred_element_type=jnp.float32)
    z = jnp.maximum(z + bg_ref[...], 0.0)
    z_ref[...] = z
    y_ref[...] = jnp.dot(z, wc_ref[...],
                         preferred_element_type=jnp.float32) + bc_ref[...]
    s_ref[...] = jnp.dot(z, ws_ref[...],
                         preferred_element_type=jnp.float32) + bs_ref[...]


def kernel(x, edge_index, W_gcn, b_gcn, W_cls, b_cls, W_sen, b_sen):
    src = edge_index[0]
    dst = edge_index[1]
    pad = NW * EPW - N_EDGES
    padv = N_NODES + jnp.arange(pad, dtype=jnp.int32) % (NP - N_NODES)
    srcr = jnp.concatenate([src, padv]).reshape(NW, CH, 128)
    dstr = jnp.concatenate([dst, padv]).reshape(NW, CH, 128)
    x_pad = jnp.pad(x, ((0, NP - N_NODES), (0, 0)))
    zeros_np = jnp.zeros((NP,), jnp.float32)
    zeros_rows = jnp.zeros((RPT, D), jnp.float32)
    ones_128 = jnp.ones((128,), jnp.float32)

    dsrc_p, ddst_p = _deg_call(srcr, dstr, zeros_np, ones_128)

    h, nd = pl.pallas_call(
        _prep_body,
        out_shape=[
            jax.ShapeDtypeStruct((NP, D), jnp.float32),
            jax.ShapeDtypeStruct((NP, 1), jnp.float32),
        ],
    )(x_pad, dsrc_p, ddst_p)

    parts = _agg_call(h, srcr.reshape(NW * CH, 128),
                      dstr.reshape(NW * CH, 128), zeros_rows)

    z, y, s = pl.pallas_call(
        _head_body,
        out_shape=[
            jax.ShapeDtypeStruct((NP, D), jnp.float32),
            jax.ShapeDtypeStruct((NP, 1), jnp.float32),
            jax.ShapeDtypeStruct((NP, 1), jnp.float32),
        ],
    )(parts, nd, W_gcn, b_gcn.reshape(1, D), W_cls, b_cls.reshape(1, 1),
      W_sen, b_sen.reshape(1, 1))

    return (z[:N_NODES], y[:N_NODES], s[:N_NODES])


# speedup vs baseline: 3.1596x; 1.0891x over previous
import functools

import jax
import jax.numpy as jnp
from jax.experimental import pallas as pl
from jax.experimental.pallas import tpu as pltpu
from jax.experimental.pallas import tpu_sc as plsc

N_NODES = 10000
N_EDGES = 320000
D = 128

NC = 2
NS = 16
NW = NC * NS
NP = 10240
CH = 80
EPW = CH * 128
RPT = NP // NS
SB = 40

_mesh = plsc.VectorSubcoreMesh(
    core_axis_name="c", subcore_axis_name="s", num_cores=NC, num_subcores=NS)


def _deg_body(srcr, dstr, zeros_np, ones_hbm, dsrc_out, ddst_out,
              sidx, didx, ones_v, hs, hd, sems, semd):
    c = jax.lax.axis_index("c")
    s = jax.lax.axis_index("s")
    w = s * NC + c
    pltpu.sync_copy(srcr.at[w], sidx)
    pltpu.sync_copy(dstr.at[w], didx)
    pltpu.sync_copy(ones_hbm, ones_v)

    @pl.when(s == 0)
    def _():
        pltpu.sync_copy(zeros_np, hs)
        pltpu.sync_copy(zeros_np, hd)

    plsc.subcore_barrier()

    def wave(b, carry):
        def fire(j, carry2):
            pltpu.async_copy(ones_v, hs.at[sidx.at[b * 16 + j]], sems, add=True)
            pltpu.async_copy(ones_v, hd.at[didx.at[b * 16 + j]], semd, add=True)
            return carry2

        jax.lax.fori_loop(0, 16, fire, 0)

        def drain(j, carry2):
            pltpu.make_async_copy(ones_v, hs.at[sidx.at[0]], sems).wait()
            pltpu.make_async_copy(ones_v, hd.at[didx.at[0]], semd).wait()
            return carry2

        jax.lax.fori_loop(0, 16, drain, 0)
        return carry

    jax.lax.fori_loop(0, CH // 16, wave, 0)
    plsc.subcore_barrier()

    @pl.when(s == 0)
    def _():
        pltpu.sync_copy(hs, dsrc_out.at[c])
        pltpu.sync_copy(hd, ddst_out.at[c])


_deg_call = functools.partial(
    pl.kernel,
    out_type=[
        jax.ShapeDtypeStruct((NC, NP), jnp.float32),
        jax.ShapeDtypeStruct((NC, NP), jnp.float32),
    ],
    mesh=_mesh,
    scratch_types=[
        pltpu.VMEM((CH, 128), jnp.int32),
        pltpu.VMEM((CH, 128), jnp.int32),
        pltpu.VMEM((128,), jnp.float32),
        pltpu.VMEM_SHARED((NP,), jnp.float32),
        pltpu.VMEM_SHARED((NP,), jnp.float32),
        pltpu.SemaphoreType.DMA,
        pltpu.SemaphoreType.DMA,
    ],
)(_deg_body)


def _agg_body(h, srcr, dstr, zrows, out, sblk, dblk, rows0, rows1, aggsh,
              sem0, sem1):
    c = jax.lax.axis_index("c")
    s = jax.lax.axis_index("s")
    pltpu.sync_copy(zrows, aggsh.at[pl.ds(s * RPT, RPT)])
    plsc.subcore_barrier()

    base = (c * NS + s) * CH

    def wait_gather(buf, sem):
        pltpu.make_async_copy(h.at[pl.ds(0, 128)], buf, sem).wait()

    def do_superchunk(row0):
        pltpu.sync_copy(srcr.at[pl.ds(row0, SB)], sblk)
        pltpu.sync_copy(dstr.at[pl.ds(row0, SB)], dblk)
        pltpu.async_copy(h.at[sblk.at[0]], rows0, sem0)

        def inner(i, carry2):
            j0 = 2 * i
            j1 = j0 + 1
            pltpu.async_copy(h.at[sblk.at[j1]], rows1, sem1)
            wait_gather(rows0, sem0)
            pltpu.sync_copy(rows0, aggsh.at[dblk.at[j0]], add=True)

            @pl.when(i < SB // 2 - 1)
            def _():
                pltpu.async_copy(h.at[sblk.at[j0 + 2]], rows0, sem0)

            wait_gather(rows1, sem1)
            pltpu.sync_copy(rows1, aggsh.at[dblk.at[j1]], add=True)
            return carry2

        jax.lax.fori_loop(0, SB // 2, inner, 0)

    def outer(b, carry):
        do_superchunk(base + b * SB)
        return carry

    jax.lax.fori_loop(0, CH // SB, outer, 0)
    plsc.subcore_barrier()
    pltpu.sync_copy(aggsh.at[pl.ds(s * RPT, RPT)],
                    out.at[c, pl.ds(s * RPT, RPT)])


_agg_call = functools.partial(
    pl.kernel,
    out_type=jax.ShapeDtypeStruct((NC, NP, D), jnp.float32),
    mesh=_mesh,
    scratch_types=[
        pltpu.VMEM((SB, 128), jnp.int32),
        pltpu.VMEM((SB, 128), jnp.int32),
        pltpu.VMEM((128, D), jnp.float32),
        pltpu.VMEM((128, D), jnp.float32),
        pltpu.VMEM_SHARED((NP, D), jnp.float32),
        pltpu.SemaphoreType.DMA,
        pltpu.SemaphoreType.DMA,
    ],
)(_agg_body)


def _prep_body(x_ref, ds_ref, dd_ref, h_ref, nd_ref):
    ds = jnp.sum(ds_ref[...], axis=0)
    dd = jnp.sum(dd_ref[...], axis=0)
    ns = jax.lax.rsqrt(jnp.maximum(ds, 1.0))
    h_ref[...] = x_ref[...] * ns[:, None]
    nd_ref[...] = jax.lax.rsqrt(jnp.maximum(dd, 1.0)).reshape(NP, 1)


def _head_body(p_ref, nd_ref, wg_ref, bg_ref, wc_ref, bc_ref, ws_ref, bs_ref,
               z_ref, y_ref, s_ref):
    agg = ((p_ref[0, :N_NODES] + p_ref[1, :N_NODES])
           * nd_ref[...][:N_NODES])
    z = jnp.dot(agg, wg_ref[...], preferred_element_type=jnp.float32)
    z = jnp.maximum(z + bg_ref[...], 0.0)
    z_ref[...] = z
    y_ref[...] = jnp.dot(z, wc_ref[...],
                         preferred_element_type=jnp.float32) + bc_ref[...]
    s_ref[...] = jnp.dot(z, ws_ref[...],
                         preferred_element_type=jnp.float32) + bs_ref[...]


def kernel(x, edge_index, W_gcn, b_gcn, W_cls, b_cls, W_sen, b_sen):
    src = edge_index[0]
    dst = edge_index[1]
    pad = NW * EPW - N_EDGES
    padv = N_NODES + jnp.arange(pad, dtype=jnp.int32) % (NP - N_NODES)
    srcr = jnp.concatenate([src, padv]).reshape(NW, CH, 128)
    dstr = jnp.concatenate([dst, padv]).reshape(NW, CH, 128)
    x_pad = jnp.pad(x, ((0, NP - N_NODES), (0, 0)))
    zeros_np = jnp.zeros((NP,), jnp.float32)
    zeros_rows = jnp.zeros((RPT, D), jnp.float32)
    ones_128 = jnp.ones((128,), jnp.float32)

    dsrc_p, ddst_p = _deg_call(srcr, dstr, zeros_np, ones_128)

    h, nd = pl.pallas_call(
        _prep_body,
        out_shape=[
            jax.ShapeDtypeStruct((NP, D), jnp.float32),
            jax.ShapeDtypeStruct((NP, 1), jnp.float32),
        ],
    )(x_pad, dsrc_p, ddst_p)

    parts = _agg_call(h, srcr.reshape(NW * CH, 128),
                      dstr.reshape(NW * CH, 128), zeros_rows)

    z, y, s = pl.pallas_call(
        _head_body,
        out_shape=[
            jax.ShapeDtypeStruct((N_NODES, D), jnp.float32),
            jax.ShapeDtypeStruct((N_NODES, 1), jnp.float32),
            jax.ShapeDtypeStruct((N_NODES, 1), jnp.float32),
        ],
    )(parts, nd, W_gcn, b_gcn.reshape(1, D), W_cls, b_cls.reshape(1, 1),
      W_sen, b_sen.reshape(1, 1))

    return (z, y, s)
